# baseline (device time: 66331 ns/iter reference)
import jax
import jax.numpy as jnp
from jax import lax
from jax.experimental import pallas as pl
from jax.experimental.pallas import tpu as pltpu

N_DEV = 8
E_LOCAL = 8

SECS = ((0, 384, (0, 1, 2)), (384, 384, (1, 2, 0)), (768, 256, (2, 0, 1)))


def _xor(a, b):
    return a + b - 2 * a * b


def _pos(bx, by, bz):
    return bz * 4 + 2 * by + _xor(bx, by)


def kernel(x, router_W, route_idx, expert_W, shared_W):
    n, d = x.shape
    _, h = shared_W.shape
    rows = n // N_DEV

    def body(x_ref, rw_ref, idx_ref, ew_ref, sw_ref, out_ref,
             ptok_ref, ew16_ref, acc_a, acc_b, acc_c,
             sb1a, sb1b, sb1c, sb2a, sb2b, sb2c, sb3a, sb3b, sb3c,
             r1a, r1b, r1c, r2a, r2b, r2c, r3a, r3b, r3c,
             send_sems, recv_sems):
        accs = (acc_a, acc_b, acc_c)
        sbufs = ((sb1a, sb2a, sb3a), (sb1b, sb2b, sb3b), (sb1c, sb2c, sb3c))
        rv = ((r1a, r2a, r3a), (r1b, r2b, r3b), (r1c, r2c, r3c))

        p = lax.axis_index("i")
        mz = p // 4
        q = lax.rem(p, 4)
        myy = q // 2
        mx = _xor(lax.rem(q, 2), myy)
        mb = (mx, myy, mz)

        def flipped(dim):
            bits = list(mb)
            bits[dim] = 1 - bits[dim]
            return _pos(*bits)

        partner = [flipped(0), flipped(1), flipped(2)]

        barrier_sem = pltpu.get_barrier_semaphore()
        for dim in range(3):
            pl.semaphore_signal(
                barrier_sem, inc=1,
                device_id=(partner[dim],),
                device_id_type=pl.DeviceIdType.MESH,
            )
        pl.semaphore_wait(barrier_sem, 3)

        xv = x_ref[...]
        scores = jnp.dot(xv, rw_ref[...], preferred_element_type=jnp.float32)
        scores = scores - jnp.max(scores, axis=1, keepdims=True)
        e_sc = jnp.exp(scores)
        probs = e_sc / jnp.sum(e_sc, axis=1, keepdims=True)
        ids = idx_ref[...]
        eids = lax.broadcasted_iota(jnp.int32, probs.shape, 1)
        ptok_ref[...] = jnp.sum(jnp.where(eids == ids, probs, 0.0),
                                axis=1, keepdims=True)

        ew16_ref[...] = ew_ref[...].astype(jnp.bfloat16)

        def partial_val(c, sec):
            off, w, _ = SECS[sec]
            xc = x_ref[pl.ds(c * rows, rows), :]
            idc = idx_ref[pl.ds(c * rows, rows), :]
            pc = ptok_ref[pl.ds(c * rows, rows), :]
            xpc = (xc * pc).astype(jnp.bfloat16)
            zero = jnp.zeros_like(xpc)
            acc = None
            for j in range(E_LOCAL):
                e = p * E_LOCAL + j
                xm = jnp.where(idc == e, xpc, zero)
                g = jnp.dot(xm, ew16_ref[j, :, pl.ds(off, w)],
                            preferred_element_type=jnp.float32)
                acc = g if acc is None else acc + g
            return acc

        def send_bits(sec, phase, slot):
            d1, d2, d3 = SECS[sec][2]
            b = [None] * 3
            if phase == 1:
                b[d1] = 1 - mb[d1]
                b[d2] = slot % 2
                b[d3] = slot // 2
            elif phase == 2:
                b[d1] = mb[d1]
                b[d2] = 1 - mb[d2]
                b[d3] = slot
            else:
                b[d1], b[d2], b[d3] = mb[d1], mb[d2], 1 - mb[d3]
            return b

        msgs = {}

        def start_msg(sec, phase):
            order = SECS[sec][2]
            mid = sec * 3 + (phase - 1)
            desc = pltpu.make_async_remote_copy(
                src_ref=sbufs[sec][phase - 1],
                dst_ref=rv[sec][phase - 1],
                send_sem=send_sems.at[mid],
                recv_sem=recv_sems.at[mid],
                device_id=(partner[order[phase - 1]],),
                device_id_type=pl.DeviceIdType.MESH,
            )
            desc.start()
            msgs[(sec, phase)] = desc

        def r_slot(sec, phase, slot):
            return rv[sec][phase - 1][pl.ds(slot, 1)][0]

        for sec in range(3):
            for slot in range(4):
                sbufs[sec][0][slot] = partial_val(
                    _pos(*send_bits(sec, 1, slot)), sec)
            start_msg(sec, 1)

        for sec in range(3):
            for slot in range(2):
                c = _pos(*send_bits(sec, 2, slot))
                accs[sec][pl.ds(c * rows, rows), :] = partial_val(c, sec)

        for sec in range(3):
            d1, d2, d3 = SECS[sec][2]
            msgs[(sec, 1)].wait_recv()
            for v in range(2):
                c = _pos(*send_bits(sec, 2, v))
                s1 = 2 * v + (1 - mb[d2])
                sbufs[sec][1][v] = (accs[sec][pl.ds(c * rows, rows), :]
                                    + r_slot(sec, 1, s1))
            start_msg(sec, 2)

        for sec in range(3):
            d1, d2, d3 = SECS[sec][2]
            for v in range(2):
                bits = [None] * 3
                bits[d1], bits[d2], bits[d3] = mb[d1], mb[d2], v
                c = _pos(*bits)
                accs[sec][pl.ds(c * rows, rows), :] = partial_val(c, sec)

        for sec in range(3):
            d1, d2, d3 = SECS[sec][2]
            msgs[(sec, 2)].wait_recv()
            c = _pos(*send_bits(sec, 3, 0))
            s1 = 2 * (1 - mb[d3]) + mb[d2]
            s2 = 1 - mb[d3]
            sbufs[sec][2][0] = (accs[sec][pl.ds(c * rows, rows), :]
                                + r_slot(sec, 1, s1) + r_slot(sec, 2, s2))
            start_msg(sec, 3)

        xs = x_ref[pl.ds(p * rows, rows), :]
        shared = jnp.dot(xs, sw_ref[...], preferred_element_type=jnp.float32)

        for sec in range(3):
            d1, d2, d3 = SECS[sec][2]
            msgs[(sec, 3)].wait_recv()
            off, w, _ = SECS[sec]
            s1 = 2 * mb[d3] + mb[d2]
            s2 = mb[d3]
            out_ref[:, pl.ds(off, w)] = (
                accs[sec][pl.ds(p * rows, rows), :]
                + r_slot(sec, 1, s1) + r_slot(sec, 2, s2)
                + rv[sec][2][0] + shared[:, off:off + w]
            )

        for desc in msgs.values():
            desc.wait_send()

    wa, wb, wc = SECS[0][1], SECS[1][1], SECS[2][1]
    return pl.pallas_call(
        body,
        out_shape=jax.ShapeDtypeStruct((rows, h), jnp.float32),
        in_specs=[pl.BlockSpec(memory_space=pltpu.VMEM)] * 5,
        out_specs=pl.BlockSpec(memory_space=pltpu.VMEM),
        scratch_shapes=[
            pltpu.VMEM((n, 1), jnp.float32),
            pltpu.VMEM((E_LOCAL, d, h), jnp.bfloat16),
            pltpu.VMEM((n, wa), jnp.float32),
            pltpu.VMEM((n, wb), jnp.float32),
            pltpu.VMEM((n, wc), jnp.float32),
            pltpu.VMEM((4, rows, wa), jnp.float32),
            pltpu.VMEM((4, rows, wb), jnp.float32),
            pltpu.VMEM((4, rows, wc), jnp.float32),
            pltpu.VMEM((2, rows, wa), jnp.float32),
            pltpu.VMEM((2, rows, wb), jnp.float32),
            pltpu.VMEM((2, rows, wc), jnp.float32),
            pltpu.VMEM((1, rows, wa), jnp.float32),
            pltpu.VMEM((1, rows, wb), jnp.float32),
            pltpu.VMEM((1, rows, wc), jnp.float32),
            pltpu.VMEM((4, rows, wa), jnp.float32),
            pltpu.VMEM((4, rows, wb), jnp.float32),
            pltpu.VMEM((4, rows, wc), jnp.float32),
            pltpu.VMEM((2, rows, wa), jnp.float32),
            pltpu.VMEM((2, rows, wb), jnp.float32),
            pltpu.VMEM((2, rows, wc), jnp.float32),
            pltpu.VMEM((1, rows, wa), jnp.float32),
            pltpu.VMEM((1, rows, wb), jnp.float32),
            pltpu.VMEM((1, rows, wc), jnp.float32),
            pltpu.SemaphoreType.DMA((9,)),
            pltpu.SemaphoreType.DMA((9,)),
        ],
        compiler_params=pltpu.CompilerParams(
            collective_id=0, vmem_limit_bytes=62 * 1024 * 1024
        ),
    )(x, router_W, route_idx, expert_W, shared_W)


# device time: 52311 ns/iter; 1.2680x vs baseline; 1.2680x over previous
import jax
import jax.numpy as jnp
from jax import lax
from jax.experimental import pallas as pl
from jax.experimental.pallas import tpu as pltpu

N_DEV = 8
E_LOCAL = 8

SECS = ((0, 384, (0, 1, 2)), (384, 384, (1, 2, 0)), (768, 256, (2, 0, 1)))


def _xor(a, b):
    return a + b - 2 * a * b


def _pos(bx, by, bz):
    return bz * 4 + 2 * by + _xor(bx, by)


def kernel(x, router_W, route_idx, expert_W, shared_W):
    n, d = x.shape
    _, h = shared_W.shape
    rows = n // N_DEV

    def body(x_ref, rw_ref, idx_ref, ew_ref, sw_ref, out_ref,
             ptok_ref, ew16_ref, acc_a, acc_b, acc_c,
             sb1a, sb1b, sb1c, sb2a, sb2b, sb2c, sb3a, sb3b, sb3c,
             r1a, r1b, r1c, r2a, r2b, r2c, r3a, r3b, r3c,
             send_sems, recv_sems):
        accs = (acc_a, acc_b, acc_c)
        sbufs = ((sb1a, sb2a, sb3a), (sb1b, sb2b, sb3b), (sb1c, sb2c, sb3c))
        rv = ((r1a, r2a, r3a), (r1b, r2b, r3b), (r1c, r2c, r3c))

        p = lax.axis_index("i")
        mz = p // 4
        q = lax.rem(p, 4)
        myy = q // 2
        mx = _xor(lax.rem(q, 2), myy)
        mb = (mx, myy, mz)

        def flipped(dim):
            bits = list(mb)
            bits[dim] = 1 - bits[dim]
            return _pos(*bits)

        partner = [flipped(0), flipped(1), flipped(2)]

        barrier_sem = pltpu.get_barrier_semaphore()
        for dim in range(3):
            pl.semaphore_signal(
                barrier_sem, inc=1,
                device_id=(partner[dim],),
                device_id_type=pl.DeviceIdType.MESH,
            )
        pl.semaphore_wait(barrier_sem, 3)

        xv = x_ref[...]
        scores = jnp.dot(xv, rw_ref[...], preferred_element_type=jnp.float32)
        scores = scores - jnp.max(scores, axis=1, keepdims=True)
        e_sc = jnp.exp(scores)
        probs = e_sc / jnp.sum(e_sc, axis=1, keepdims=True)
        ids = idx_ref[...]
        eids = lax.broadcasted_iota(jnp.int32, probs.shape, 1)
        ptok_ref[...] = jnp.sum(jnp.where(eids == ids, probs, 0.0),
                                axis=1, keepdims=True)

        ew16_ref[...] = ew_ref[...].astype(jnp.bfloat16)

        def partial_val(c, sec):
            off, w, _ = SECS[sec]
            xc = x_ref[pl.ds(c * rows, rows), :]
            idc = idx_ref[pl.ds(c * rows, rows), :]
            pc = ptok_ref[pl.ds(c * rows, rows), :]
            xpc = (xc * pc).astype(jnp.bfloat16)
            zero = jnp.zeros_like(xpc)
            acc = None
            for j in range(E_LOCAL):
                e = p * E_LOCAL + j
                xm = jnp.where(idc == e, xpc, zero)
                g = jnp.dot(xm, ew16_ref[j, :, pl.ds(off, w)],
                            preferred_element_type=jnp.float32)
                acc = g if acc is None else acc + g
            return acc

        def send_bits(sec, phase, slot):
            d1, d2, d3 = SECS[sec][2]
            b = [None] * 3
            if phase == 1:
                b[d1] = 1 - mb[d1]
                b[d2] = slot % 2
                b[d3] = slot // 2
            elif phase == 2:
                b[d1] = mb[d1]
                b[d2] = 1 - mb[d2]
                b[d3] = slot
            else:
                b[d1], b[d2], b[d3] = mb[d1], mb[d2], 1 - mb[d3]
            return b

        msgs = {}

        def start_msg(sec, phase):
            order = SECS[sec][2]
            mid = sec * 3 + (phase - 1)
            desc = pltpu.make_async_remote_copy(
                src_ref=sbufs[sec][phase - 1],
                dst_ref=rv[sec][phase - 1],
                send_sem=send_sems.at[mid],
                recv_sem=recv_sems.at[mid],
                device_id=(partner[order[phase - 1]],),
                device_id_type=pl.DeviceIdType.MESH,
            )
            desc.start()
            msgs[(sec, phase)] = desc

        def r_slot(sec, phase, slot):
            return rv[sec][phase - 1][pl.ds(slot, 1)][0]

        for sec in range(3):
            for slot in range(4):
                sbufs[sec][0][slot] = partial_val(
                    _pos(*send_bits(sec, 1, slot)), sec
                ).astype(jnp.bfloat16)
            start_msg(sec, 1)

        for sec in range(3):
            for slot in range(2):
                c = _pos(*send_bits(sec, 2, slot))
                accs[sec][pl.ds(c * rows, rows), :] = partial_val(c, sec)

        for sec in range(3):
            d1, d2, d3 = SECS[sec][2]
            msgs[(sec, 1)].wait_recv()
            for v in range(2):
                c = _pos(*send_bits(sec, 2, v))
                s1 = 2 * v + (1 - mb[d2])
                sbufs[sec][1][v] = (accs[sec][pl.ds(c * rows, rows), :]
                                    + r_slot(sec, 1, s1)
                                    ).astype(jnp.bfloat16)
            start_msg(sec, 2)

        for sec in range(3):
            d1, d2, d3 = SECS[sec][2]
            for v in range(2):
                bits = [None] * 3
                bits[d1], bits[d2], bits[d3] = mb[d1], mb[d2], v
                c = _pos(*bits)
                accs[sec][pl.ds(c * rows, rows), :] = partial_val(c, sec)

        for sec in range(3):
            d1, d2, d3 = SECS[sec][2]
            msgs[(sec, 2)].wait_recv()
            c = _pos(*send_bits(sec, 3, 0))
            s1 = 2 * (1 - mb[d3]) + mb[d2]
            s2 = 1 - mb[d3]
            sbufs[sec][2][0] = (accs[sec][pl.ds(c * rows, rows), :]
                                + r_slot(sec, 1, s1) + r_slot(sec, 2, s2)
                                ).astype(jnp.bfloat16)
            start_msg(sec, 3)

        xs = x_ref[pl.ds(p * rows, rows), :]
        shared = jnp.dot(xs, sw_ref[...], preferred_element_type=jnp.float32)

        for sec in range(3):
            d1, d2, d3 = SECS[sec][2]
            msgs[(sec, 3)].wait_recv()
            off, w, _ = SECS[sec]
            s1 = 2 * mb[d3] + mb[d2]
            s2 = mb[d3]
            out_ref[:, pl.ds(off, w)] = (
                accs[sec][pl.ds(p * rows, rows), :]
                + r_slot(sec, 1, s1) + r_slot(sec, 2, s2)
                + rv[sec][2][0] + shared[:, off:off + w]
            )

        for desc in msgs.values():
            desc.wait_send()

    wa, wb, wc = SECS[0][1], SECS[1][1], SECS[2][1]
    return pl.pallas_call(
        body,
        out_shape=jax.ShapeDtypeStruct((rows, h), jnp.float32),
        in_specs=[pl.BlockSpec(memory_space=pltpu.VMEM)] * 5,
        out_specs=pl.BlockSpec(memory_space=pltpu.VMEM),
        scratch_shapes=[
            pltpu.VMEM((n, 1), jnp.float32),
            pltpu.VMEM((E_LOCAL, d, h), jnp.bfloat16),
            pltpu.VMEM((n, wa), jnp.float32),
            pltpu.VMEM((n, wb), jnp.float32),
            pltpu.VMEM((n, wc), jnp.float32),
            pltpu.VMEM((4, rows, wa), jnp.bfloat16),
            pltpu.VMEM((4, rows, wb), jnp.bfloat16),
            pltpu.VMEM((4, rows, wc), jnp.bfloat16),
            pltpu.VMEM((2, rows, wa), jnp.bfloat16),
            pltpu.VMEM((2, rows, wb), jnp.bfloat16),
            pltpu.VMEM((2, rows, wc), jnp.bfloat16),
            pltpu.VMEM((1, rows, wa), jnp.bfloat16),
            pltpu.VMEM((1, rows, wb), jnp.bfloat16),
            pltpu.VMEM((1, rows, wc), jnp.bfloat16),
            pltpu.VMEM((4, rows, wa), jnp.bfloat16),
            pltpu.VMEM((4, rows, wb), jnp.bfloat16),
            pltpu.VMEM((4, rows, wc), jnp.bfloat16),
            pltpu.VMEM((2, rows, wa), jnp.bfloat16),
            pltpu.VMEM((2, rows, wb), jnp.bfloat16),
            pltpu.VMEM((2, rows, wc), jnp.bfloat16),
            pltpu.VMEM((1, rows, wa), jnp.bfloat16),
            pltpu.VMEM((1, rows, wb), jnp.bfloat16),
            pltpu.VMEM((1, rows, wc), jnp.bfloat16),
            pltpu.SemaphoreType.DMA((9,)),
            pltpu.SemaphoreType.DMA((9,)),
        ],
        compiler_params=pltpu.CompilerParams(
            collective_id=0, vmem_limit_bytes=62 * 1024 * 1024
        ),
    )(x, router_W, route_idx, expert_W, shared_W)
